# baseline (speedup 1.0000x reference)
import functools

import jax
import jax.numpy as jnp
from jax import lax
from jax.experimental import pallas as pl
from jax.experimental.pallas import tpu as pltpu
from jax.experimental.pallas import tpu_sc as plsc

NCORES = 2
NSUB = 16
NW = NCORES * NSUB
L = 16

LAST_TIME = 1.0
INIT_TIME = 0.0


def _build_table(x0_t, v_t, n, d, bins, bw, nb):
    rows = (bins + 1) * d
    grid = (n + nb - 1) // nb

    def body(x0_ref, v_ref, out_ref, s_ref):
        acc = x0_ref[...]
        for b in range(bins + 1):
            s_ref[b * d:(b + 1) * d, :] = acc
            if b < bins:
                acc = acc + bw * v_ref[b]
        out_ref[...] = s_ref[...].T

    return pl.pallas_call(
        body,
        grid=(grid,),
        in_specs=[
            pl.BlockSpec((d, nb), lambda g: (0, g)),
            pl.BlockSpec((bins, d, nb), lambda g: (0, 0, g)),
        ],
        out_specs=pl.BlockSpec((nb, rows), lambda g: (g, 0)),
        out_shape=jax.ShapeDtypeStruct((n, rows), jnp.float32),
        scratch_shapes=[pltpu.VMEM((rows, nb), jnp.float32)],
        compiler_params=pltpu.CompilerParams(
            dimension_semantics=("parallel",),
            vmem_limit_bytes=66977792),
    )(x0_t, v_t)


def _sc_intensity(a_rows, beta, times, ii, jj, n, d, bins, bw, p):
    cpw = p // NW
    ch = 128
    nch = cpw // ch
    binsm1 = jnp.int32(bins - 1)
    nb1 = jnp.int32(bins + 1)
    inv_bw = float(1.0 / bw)

    mesh = plsc.VectorSubcoreMesh(core_axis_name="c", subcore_axis_name="s")

    def _buf_set():
        return [
            pltpu.VMEM((ch,), jnp.int32),
            pltpu.VMEM((ch,), jnp.int32),
            pltpu.VMEM((ch,), jnp.int32),
            pltpu.VMEM((ch,), jnp.int32),
            pltpu.VMEM((ch,), jnp.int32),
            pltpu.VMEM((ch,), jnp.int32),
            pltpu.VMEM((ch,), jnp.float32),
            pltpu.VMEM((ch, d), jnp.float32),
            pltpu.VMEM((ch, d), jnp.float32),
            pltpu.VMEM((ch, d), jnp.float32),
            pltpu.VMEM((ch, d), jnp.float32),
            pltpu.VMEM((ch,), jnp.float32),
            pltpu.VMEM((ch,), jnp.float32),
            pltpu.SemaphoreType.DMA,
        ]

    @functools.partial(
        pl.kernel,
        out_type=jax.ShapeDtypeStruct((p,), jnp.float32),
        mesh=mesh,
        compiler_params=pltpu.CompilerParams(use_tc_tiling_on_sc=False),
        scratch_types=[
            pltpu.VMEM((cpw,), jnp.float32),
            pltpu.VMEM((cpw,), jnp.int32),
            pltpu.VMEM((cpw,), jnp.int32),
            pltpu.VMEM((cpw,), jnp.float32),
        ] + _buf_set() + _buf_set(),
    )
    def k(a_hbm, b_hbm, t_hbm, i_hbm, j_hbm, out_hbm,
          t_v, i_v, j_v, out_v, *bufs):
        s0, s1 = bufs[:14], bufs[14:]
        wid = lax.axis_index("s") * NCORES + lax.axis_index("c")
        base = pl.multiple_of(wid * cpw, cpw)
        pltpu.sync_copy(t_hbm.at[pl.ds(base, cpw)], t_v)
        pltpu.sync_copy(i_hbm.at[pl.ds(base, cpw)], i_v)
        pltpu.sync_copy(j_hbm.at[pl.ds(base, cpw)], j_v)
        lane = lax.iota(jnp.int32, L)

        def idx_fire(c0, S):
            gi0_v, gi1_v, gj0_v, gj1_v, ic_v, jc_v, s_v = S[:7]
            ai0_v, ai1_v, aj0_v, aj1_v, bi_v, bj_v, sem = S[7:]
            for g in range(ch // L):
                src = pl.ds(c0 + g * L, L)
                dst = pl.ds(g * L, L)
                t16 = t_v[src]
                i16 = i_v[src]
                j16 = j_v[src]
                q = t16 * inv_bw
                bidx = q.astype(jnp.int32)
                s16 = q - bidx.astype(jnp.float32)
                b0 = jnp.minimum(bidx, binsm1)
                ri = i16 * nb1 + b0
                rj = j16 * nb1 + b0
                gi0_v[dst] = ri
                gi1_v[dst] = ri + 1
                gj0_v[dst] = rj
                gj1_v[dst] = rj + 1
                ic_v[dst] = i16
                jc_v[dst] = j16
                s_v[dst] = s16
            pltpu.async_copy(a_hbm.at[gi0_v], ai0_v, sem)
            pltpu.async_copy(a_hbm.at[gi1_v], ai1_v, sem)
            pltpu.async_copy(a_hbm.at[gj0_v], aj0_v, sem)
            pltpu.async_copy(a_hbm.at[gj1_v], aj1_v, sem)
            pltpu.async_copy(b_hbm.at[ic_v], bi_v, sem)
            pltpu.async_copy(b_hbm.at[jc_v], bj_v, sem)

        def wait_all(S):
            gi0_v, gi1_v, gj0_v, gj1_v, ic_v, jc_v, s_v = S[:7]
            ai0_v, ai1_v, aj0_v, aj1_v, bi_v, bj_v, sem = S[7:]
            pltpu.make_async_copy(a_hbm.at[gi0_v], ai0_v, sem).wait()
            pltpu.make_async_copy(a_hbm.at[gi1_v], ai1_v, sem).wait()
            pltpu.make_async_copy(a_hbm.at[gj0_v], aj0_v, sem).wait()
            pltpu.make_async_copy(a_hbm.at[gj1_v], aj1_v, sem).wait()
            pltpu.make_async_copy(b_hbm.at[ic_v], bi_v, sem).wait()
            pltpu.make_async_copy(b_hbm.at[jc_v], bj_v, sem).wait()

        def compute(c0, S):
            s_v = S[6]
            ai0_v, ai1_v, aj0_v, aj1_v, bi_v, bj_v = S[7:13]

            def group(g2, carry2):
                gb2 = pl.multiple_of(g2 * L, L)
                sl = pl.ds(gb2, L)
                s16 = s_v[sl]
                ys = []
                for u in range(L):
                    pp = gb2 + u
                    d0 = ai0_v[pp, :] - aj0_v[pp, :]
                    d1 = ai1_v[pp, :] - aj1_v[pp, :]
                    xt = d0 + s16[u] * (d1 - d0)
                    ys.append(xt * xt)
                kk = 1
                while len(ys) > 1:
                    mask = (lane & kk) == 0
                    ys = [
                        jnp.where(mask, a, b)
                        + jnp.where(mask, b, a).at[lane ^ kk].get(
                            mode="promise_in_bounds")
                        for a, b in zip(ys[0::2], ys[1::2])
                    ]
                    kk <<= 1
                o16 = jnp.exp(bi_v[sl] + bj_v[sl] - ys[0])
                out_v[pl.ds(c0 + gb2, L)] = o16
                return carry2

            lax.fori_loop(0, ch // L, group, 0, unroll=2)

        idx_fire(0, s0)

        def pair_body(h, carry):
            c_even = pl.multiple_of(h * (2 * ch), 2 * ch)
            c_odd = c_even + ch
            idx_fire(c_odd, s1)
            wait_all(s0)
            compute(c_even, s0)

            @pl.when(h < nch // 2 - 1)
            def _():
                idx_fire(c_even + 2 * ch, s0)

            wait_all(s1)
            compute(c_odd, s1)
            return carry

        lax.fori_loop(0, nch // 2, pair_body, 0)
        pltpu.sync_copy(out_v, out_hbm.at[pl.ds(base, cpw)])

    return k(a_rows, beta, times, ii, jj)


def kernel(x0, v, beta, times_list, node_pairs):
    n, d = x0.shape
    bins = v.shape[0]
    p = times_list.shape[0]
    bw = (LAST_TIME - INIT_TIME) / float(bins)

    x0_t = x0.T
    v_t = v.transpose(0, 2, 1)
    a3 = _build_table(x0_t, v_t, n, d, bins, bw, 8192)
    a_rows = a3.reshape(n * (bins + 1), d)
    return _sc_intensity(a_rows, beta, times_list,
                         node_pairs[0], node_pairs[1], n, d, bins, bw, p)


# speedup vs baseline: 2.0688x; 2.0688x over previous
import functools

import jax
import jax.numpy as jnp
from jax import lax
from jax.experimental import pallas as pl
from jax.experimental.pallas import tpu as pltpu
from jax.experimental.pallas import tpu_sc as plsc

NCORES = 2
NSUB = 16
NW = NCORES * NSUB
L = 16

LAST_TIME = 1.0
INIT_TIME = 0.0


def _build_table(x0_t, v_t, n, d, bins, bw, nb, gsz, ngrp):
    grid = (n + nb - 1) // nb

    def body(x0_ref, v_ref, out_ref, s_ref):
        acc = x0_ref[...]
        for bg in range(ngrp):
            for bl in range(gsz):
                b = bg * gsz + bl
                if b <= bins:
                    s_ref[bl * d:(bl + 1) * d, :] = acc
                    if b < bins:
                        acc = acc + bw * v_ref[b]
            out_ref[bg] = s_ref[...].T

    return pl.pallas_call(
        body,
        grid=(grid,),
        in_specs=[
            pl.BlockSpec((d, nb), lambda g: (0, g)),
            pl.BlockSpec((bins, d, nb), lambda g: (0, 0, g)),
        ],
        out_specs=pl.BlockSpec((ngrp, nb, 128), lambda g: (0, g, 0)),
        out_shape=jax.ShapeDtypeStruct((ngrp, n, 128), jnp.float32),
        scratch_shapes=[pltpu.VMEM((gsz * d, nb), jnp.float32)],
        compiler_params=pltpu.CompilerParams(
            dimension_semantics=("parallel",),
            vmem_limit_bytes=66977792),
    )(x0_t, v_t)


def _sc_intensity(a_rows, beta, times, ii, jj, n, d, bins, bw, p):
    cpw = p // NW
    ch = 128
    nch = cpw // ch
    binsm1 = jnp.int32(bins - 1)
    n32 = jnp.int32(n)
    inv_bw = float(1.0 / bw)

    mesh = plsc.VectorSubcoreMesh(core_axis_name="c", subcore_axis_name="s")

    def _buf_set():
        return [
            pltpu.VMEM((ch,), jnp.int32),
            pltpu.VMEM((ch,), jnp.int32),
            pltpu.VMEM((ch,), jnp.int32),
            pltpu.VMEM((ch,), jnp.int32),
            pltpu.VMEM((ch,), jnp.int32),
            pltpu.VMEM((ch,), jnp.int32),
            pltpu.VMEM((ch,), jnp.float32),
            pltpu.VMEM((ch, d), jnp.float32),
            pltpu.VMEM((ch, d), jnp.float32),
            pltpu.VMEM((ch, d), jnp.float32),
            pltpu.VMEM((ch, d), jnp.float32),
            pltpu.VMEM((ch,), jnp.float32),
            pltpu.VMEM((ch,), jnp.float32),
            pltpu.SemaphoreType.DMA,
        ]

    @functools.partial(
        pl.kernel,
        out_type=jax.ShapeDtypeStruct((p,), jnp.float32),
        mesh=mesh,
        compiler_params=pltpu.CompilerParams(use_tc_tiling_on_sc=False),
        scratch_types=[
            pltpu.VMEM((cpw,), jnp.float32),
            pltpu.VMEM((cpw,), jnp.int32),
            pltpu.VMEM((cpw,), jnp.int32),
            pltpu.VMEM((cpw,), jnp.float32),
        ] + _buf_set() + _buf_set(),
    )
    def k(a_hbm, b_hbm, t_hbm, i_hbm, j_hbm, out_hbm,
          t_v, i_v, j_v, out_v, *bufs):
        s0, s1 = bufs[:14], bufs[14:]
        wid = lax.axis_index("s") * NCORES + lax.axis_index("c")
        base = pl.multiple_of(wid * cpw, cpw)
        pltpu.sync_copy(t_hbm.at[pl.ds(base, cpw)], t_v)
        pltpu.sync_copy(i_hbm.at[pl.ds(base, cpw)], i_v)
        pltpu.sync_copy(j_hbm.at[pl.ds(base, cpw)], j_v)
        lane = lax.iota(jnp.int32, L)

        def idx_fire(c0, S):
            gi0_v, gi1_v, gj0_v, gj1_v, ic_v, jc_v, s_v = S[:7]
            ai0_v, ai1_v, aj0_v, aj1_v, bi_v, bj_v, sem = S[7:]
            for g in range(ch // L):
                src = pl.ds(c0 + g * L, L)
                dst = pl.ds(g * L, L)
                t16 = t_v[src]
                i16 = i_v[src]
                j16 = j_v[src]
                q = t16 * inv_bw
                bidx = q.astype(jnp.int32)
                s16 = q - bidx.astype(jnp.float32)
                b0 = jnp.minimum(bidx, binsm1)
                b1 = b0 + 1
                g0 = (b0 >> 3) * n32
                g1 = (b1 >> 3) * n32
                r0 = b0 & 7
                r1 = b1 & 7
                gi0_v[dst] = ((g0 + i16) << 3) + r0
                gi1_v[dst] = ((g1 + i16) << 3) + r1
                gj0_v[dst] = ((g0 + j16) << 3) + r0
                gj1_v[dst] = ((g1 + j16) << 3) + r1
                ic_v[dst] = i16
                jc_v[dst] = j16
                s_v[dst] = s16
            pltpu.async_copy(a_hbm.at[gi0_v], ai0_v, sem)
            pltpu.async_copy(a_hbm.at[gi1_v], ai1_v, sem)
            pltpu.async_copy(a_hbm.at[gj0_v], aj0_v, sem)
            pltpu.async_copy(a_hbm.at[gj1_v], aj1_v, sem)
            pltpu.async_copy(b_hbm.at[ic_v], bi_v, sem)
            pltpu.async_copy(b_hbm.at[jc_v], bj_v, sem)

        def wait_all(S):
            gi0_v, gi1_v, gj0_v, gj1_v, ic_v, jc_v, s_v = S[:7]
            ai0_v, ai1_v, aj0_v, aj1_v, bi_v, bj_v, sem = S[7:]
            pltpu.make_async_copy(a_hbm.at[gi0_v], ai0_v, sem).wait()
            pltpu.make_async_copy(a_hbm.at[gi1_v], ai1_v, sem).wait()
            pltpu.make_async_copy(a_hbm.at[gj0_v], aj0_v, sem).wait()
            pltpu.make_async_copy(a_hbm.at[gj1_v], aj1_v, sem).wait()
            pltpu.make_async_copy(b_hbm.at[ic_v], bi_v, sem).wait()
            pltpu.make_async_copy(b_hbm.at[jc_v], bj_v, sem).wait()

        def compute(c0, S):
            s_v = S[6]
            ai0_v, ai1_v, aj0_v, aj1_v, bi_v, bj_v = S[7:13]

            def group(g2, carry2):
                gb2 = pl.multiple_of(g2 * L, L)
                sl = pl.ds(gb2, L)
                s16 = s_v[sl]
                ys = []
                for u in range(L):
                    pp = gb2 + u
                    d0 = ai0_v[pp, :] - aj0_v[pp, :]
                    d1 = ai1_v[pp, :] - aj1_v[pp, :]
                    xt = d0 + s16[u] * (d1 - d0)
                    ys.append(xt * xt)
                kk = 1
                while len(ys) > 1:
                    mask = (lane & kk) == 0
                    ys = [
                        jnp.where(mask, a, b)
                        + jnp.where(mask, b, a).at[lane ^ kk].get(
                            mode="promise_in_bounds")
                        for a, b in zip(ys[0::2], ys[1::2])
                    ]
                    kk <<= 1
                o16 = jnp.exp(bi_v[sl] + bj_v[sl] - ys[0])
                out_v[pl.ds(c0 + gb2, L)] = o16
                return carry2

            lax.fori_loop(0, ch // L, group, 0, unroll=2)

        idx_fire(0, s0)

        def pair_body(h, carry):
            c_even = pl.multiple_of(h * (2 * ch), 2 * ch)
            c_odd = c_even + ch
            idx_fire(c_odd, s1)
            wait_all(s0)
            compute(c_even, s0)

            @pl.when(h < nch // 2 - 1)
            def _():
                idx_fire(c_even + 2 * ch, s0)

            wait_all(s1)
            compute(c_odd, s1)
            return carry

        lax.fori_loop(0, nch // 2, pair_body, 0)
        pltpu.sync_copy(out_v, out_hbm.at[pl.ds(base, cpw)])

    return k(a_rows, beta, times, ii, jj)


def kernel(x0, v, beta, times_list, node_pairs):
    n, d = x0.shape
    bins = v.shape[0]
    p = times_list.shape[0]
    bw = (LAST_TIME - INIT_TIME) / float(bins)

    x0_t = x0.T
    v_t = v.transpose(0, 2, 1)
    gsz = 128 // d
    ngrp = (bins + 1 + gsz - 1) // gsz
    a3 = _build_table(x0_t, v_t, n, d, bins, bw, 8192, gsz, ngrp)
    a_rows = a3.reshape(ngrp * n * gsz, d)
    return _sc_intensity(a_rows, beta, times_list,
                         node_pairs[0], node_pairs[1], n, d, bins, bw, p)
